# baseline (speedup 1.0000x reference)
import functools

import numpy as np
import jax
import jax.numpy as jnp
from jax import lax
from jax.experimental import pallas as pl

_BN_DIV = np.float32(np.sqrt(np.float32(1.0 + 1e-5)))
_HIGH = lax.Precision.HIGHEST
_BF = jnp.bfloat16

_LAYERS = [
    (1, 32, 1024, 80, 512, 40),
    (32, 32, 512, 40, 256, 20),
    (32, 64, 256, 20, 128, 10),
    (64, 64, 128, 10, 64, 5),
    (64, 128, 64, 5, 32, 3),
    (128, 128, 32, 3, 16, 2),
]

_TAP = [(0, 0), (1, 0), (0, 1)]


def _pad_planes(x, Ho, Wo):
    B, H, W, C = x.shape
    U, V = Ho + 1, Wo + 1
    xp = jnp.pad(x, ((0, 0), (1, 2 * Ho + 1 - H), (1, 2 * Wo + 1 - W), (0, 0)))
    xr = xp.reshape(B, U, 2, V, 2, C)
    planes = [xr[:, :, pi, :, pj, :].reshape(B, U * V, C)
              for pi in (0, 1) for pj in (0, 1)]
    return jnp.concatenate(planes, axis=-1)


def _conv_body(p4, w4, g, b, out, *, M, V, MT, gridded):
    shifts = (0, 1, V, V + 1)
    gv = g[...]
    bias = b[...]
    for m0 in range(0, M, MT):
        mt = min(MT, M - m0)
        acc = None
        for si, s in enumerate(shifts):
            if gridded:
                tap = p4[0, s + m0: s + m0 + mt, :]
            else:
                tap = p4[s + m0: s + m0 + mt, :]
            c = jnp.dot(tap, w4[si], preferred_element_type=jnp.float32)
            acc = c if acc is None else acc + c
        y = jnp.maximum(gv * acc / _BN_DIV + bias, 0.0)
        yb = y.astype(_BF)
        if gridded:
            out[0, m0:m0 + mt, :] = yb
        else:
            out[m0:m0 + mt, :] = yb


def _conv0_body(p, w, g, b, out):
    acc = jnp.dot(p[0], w[...], preferred_element_type=jnp.float32)
    y = g[...] * acc / _BN_DIV + b[...]
    out[0] = jnp.maximum(y, 0.0).astype(_BF)


def _conv_layer(x, w, g, b, idx):
    Cin, Cout, H, W, Ho, Wo = _LAYERS[idx]
    B = x.shape[0]
    U, V = Ho + 1, Wo + 1
    wt = w.transpose(2, 3, 1, 0).astype(_BF)
    g2 = g.reshape(1, Cout)
    b2 = b.reshape(1, Cout)

    if idx == 0:
        xp = jnp.pad(x[..., 0], ((0, 0), (1, 2 * Ho + 1 - H), (1, 2 * Wo + 1 - W)))
        taps = [xp[:, kh:kh + 2 * Ho:2, kw:kw + 2 * Wo:2]
                for kh in range(3) for kw in range(3)]
        patches = jnp.stack(taps, axis=-1).reshape(B, Ho * Wo, 9).astype(_BF)
        w9 = wt.reshape(9, Cout)
        MT = 2560
        out = pl.pallas_call(
            _conv0_body,
            grid=(B, (Ho * Wo) // MT),
            in_specs=[
                pl.BlockSpec((1, MT, 9), lambda i, j: (i, j, 0)),
                pl.BlockSpec((9, Cout), lambda i, j: (0, 0)),
                pl.BlockSpec((1, Cout), lambda i, j: (0, 0)),
                pl.BlockSpec((1, Cout), lambda i, j: (0, 0)),
            ],
            out_specs=pl.BlockSpec((1, MT, Cout), lambda i, j: (i, j, 0)),
            out_shape=jax.ShapeDtypeStruct((B, Ho * Wo, Cout), _BF),
        )(patches, w9, g2, b2)
        return out.reshape(B, Ho, Wo, Cout)

    p4 = _pad_planes(x, Ho, Wo)
    kmap = {(0, 0): 0, (0, 1): 1, (1, 0): 2}
    blocks = []
    for du, dv in ((0, 0), (0, 1), (1, 0), (1, 1)):
        rows = []
        for pi in (0, 1):
            for pj in (0, 1):
                if (du, pi) in kmap and (dv, pj) in kmap:
                    rows.append(wt[kmap[(du, pi)], kmap[(dv, pj)]])
                else:
                    rows.append(jnp.zeros((Cin, Cout), _BF))
        blocks.append(jnp.concatenate(rows, axis=0))
    w4 = jnp.stack(blocks, axis=0)

    gridded = idx <= 2
    MT = 512
    if gridded:
        Mg = Ho * V
        body = functools.partial(_conv_body, M=Mg, V=V, MT=MT, gridded=True)
        p4 = jnp.pad(p4, ((0, 0), (0, 8), (0, 0)))
        UVp = U * V + 8
        out = pl.pallas_call(
            body,
            grid=(B,),
            in_specs=[
                pl.BlockSpec((1, UVp, 4 * Cin), lambda i: (i, 0, 0)),
                pl.BlockSpec((4, 4 * Cin, Cout), lambda i: (0, 0, 0)),
                pl.BlockSpec((1, Cout), lambda i: (0, 0)),
                pl.BlockSpec((1, Cout), lambda i: (0, 0)),
            ],
            out_specs=pl.BlockSpec((1, Mg, Cout), lambda i: (i, 0, 0)),
            out_shape=jax.ShapeDtypeStruct((B, Mg, Cout), _BF),
        )(p4, w4, g2, b2)
        out = out.reshape(B, Ho, V, Cout)[:, :, :Wo, :]
    else:
        M = B * U * V
        body = functools.partial(_conv_body, M=M, V=V, MT=MT, gridded=False)
        p4 = jnp.pad(p4.reshape(M, 4 * Cin), ((0, 8), (0, 0)))
        out = pl.pallas_call(
            body,
            out_shape=jax.ShapeDtypeStruct((M, Cout), _BF),
        )(p4, w4, g2, b2)
        out = out.reshape(B, U, V, Cout)[:, :Ho, :Wo, :]
    return out


def _gru_vq_body(hs, wih, whh, bih, bhh, cb1, cb2, cb3,
                 zq1o, zq2o, zq3o, zsumo, i1o, losso):
    T, B, D = 16, 32, 256
    hs_v = hs[...]
    gi_all = lax.dot_general(hs_v, wih[...], (((1,), (1,)), ((), ())),
                             preferred_element_type=jnp.float32) + bih[...]
    h = jnp.zeros((B, D), jnp.float32)
    for t in range(T):
        gi = gi_all[t * B:(t + 1) * B, :]
        gh = lax.dot_general(h.astype(_BF), whh[...], (((1,), (1,)), ((), ())),
                             preferred_element_type=jnp.float32) + bhh[...]
        r = jax.nn.sigmoid(gi[:, 0:D] + gh[:, 0:D])
        z = jax.nn.sigmoid(gi[:, D:2 * D] + gh[:, D:2 * D])
        n = jnp.tanh(gi[:, 2 * D:3 * D] + r * gh[:, 2 * D:3 * D])
        h = (1.0 - z) * n + z * h

    res = h
    loss = jnp.zeros((), jnp.float32)
    outs = []
    K = 1024
    ones = jnp.ones((1, D), jnp.float32)
    ii = lax.broadcasted_iota(jnp.int32, (B, K), 1)
    for k, cb in enumerate((cb1, cb2, cb3)):
        emb = cb[...]
        embb = emb.astype(_BF)
        e2 = lax.dot_general(ones, emb * emb, (((1,), (1,)), ((), ())),
                             precision=_HIGH,
                             preferred_element_type=jnp.float32)
        z2 = jnp.sum(res * res, axis=1, keepdims=True)
        s = lax.dot_general(res.astype(_BF), embb, (((1,), (1,)), ((), ())),
                            preferred_element_type=jnp.float32)
        d = z2 + e2 - 2.0 * s
        m = jnp.min(d, axis=1, keepdims=True)
        cand = jnp.where(d == m, ii, K)
        idxc = jnp.min(cand, axis=1, keepdims=True)
        if k == 0:
            i1o[...] = idxc
        enc = (ii == idxc).astype(_BF)
        zq = lax.dot_general(enc, embb, (((1,), (0,)), ((), ())),
                             preferred_element_type=jnp.float32)
        diff = zq - res
        loss = loss + 1.25 * jnp.mean(diff * diff)
        zq = res + (zq - res)
        outs.append(zq)
        res = res - zq
    zq1o[...] = outs[0]
    zq2o[...] = outs[1]
    zq3o[...] = outs[2]
    zsumo[...] = outs[0] + outs[1] + outs[2]
    losso[...] = jnp.zeros((1, 1), jnp.float32) + loss


def kernel(speech, conv_w0, conv_w1, conv_w2, conv_w3, conv_w4, conv_w5,
           bn_g0, bn_g1, bn_g2, bn_g3, bn_g4, bn_g5,
           bn_b0, bn_b1, bn_b2, bn_b3, bn_b4, bn_b5,
           w_ih, w_hh, b_ih, b_hh, cb1, cb2, cb3):
    x = speech.astype(_BF)[..., None]
    conv_ws = [conv_w0, conv_w1, conv_w2, conv_w3, conv_w4, conv_w5]
    bn_gs = [bn_g0, bn_g1, bn_g2, bn_g3, bn_g4, bn_g5]
    bn_bs = [bn_b0, bn_b1, bn_b2, bn_b3, bn_b4, bn_b5]
    for i in range(6):
        x = _conv_layer(x, conv_ws[i], bn_gs[i], bn_bs[i], i)
    B, T, F, C = x.shape
    hs = x.transpose(1, 0, 3, 2).reshape(T * B, C * F)

    zq1, zq2, zq3, zsum, i1, lossm = pl.pallas_call(
        _gru_vq_body,
        out_shape=(
            jax.ShapeDtypeStruct((32, 256), jnp.float32),
            jax.ShapeDtypeStruct((32, 256), jnp.float32),
            jax.ShapeDtypeStruct((32, 256), jnp.float32),
            jax.ShapeDtypeStruct((32, 256), jnp.float32),
            jax.ShapeDtypeStruct((32, 1), jnp.int32),
            jax.ShapeDtypeStruct((1, 1), jnp.float32),
        ),
    )(hs, w_ih.astype(_BF), w_hh.astype(_BF),
      b_ih.reshape(1, 768), b_hh.reshape(1, 768), cb1, cb2, cb3)

    z_q_out = jnp.concatenate([zq1, zq2, zq3], axis=1)
    vq_loss = lossm[0, 0]
    codebooks = (zq1, zq2, zq3, zsum)
    return z_q_out, vq_loss, i1, codebooks


# speedup vs baseline: 7.6846x; 7.6846x over previous
import functools

import numpy as np
import jax
import jax.numpy as jnp
from jax import lax
from jax.experimental import pallas as pl

_BN_DIV = np.float32(np.sqrt(np.float32(1.0 + 1e-5)))
_HIGH = lax.Precision.HIGHEST
_BF = jnp.bfloat16

_LAYERS = [
    (1, 32, 1024, 80, 512, 40, 128),
    (32, 32, 512, 40, 256, 20, 128),
    (32, 64, 256, 20, 128, 10, 128),
    (64, 64, 128, 10, 64, 5, 64),
    (64, 128, 64, 5, 32, 3, 32),
    (128, 128, 32, 3, 16, 2, 16),
]


def _banded_weights(w, idx):
    Cin, Cout, H, W, Ho, Wo, _ = _LAYERS[idx]
    last = idx == 5
    Wi, Wout = W + 2, (2 if last else Wo + 2)
    wt = w.transpose(2, 3, 1, 0)
    sel = np.zeros((3, Wi, Wout), np.float32)
    for kw in range(3):
        for wo in range(Wo):
            sel[kw, 2 * wo + kw, wo if last else wo + 1] = 1.0
    sel = jnp.asarray(sel)
    bws = []
    for kh in range(3):
        bw4 = jnp.einsum('kio,kcd->icod', sel, wt[kh])
        if last:
            bw4 = bw4.transpose(0, 1, 3, 2)
        bws.append(bw4.reshape(Wi * Cin, Wout * Cout))
    return jnp.stack(bws, axis=0).astype(_BF)


def _bn_lanes(g, b, idx):
    Cin, Cout, H, W, Ho, Wo, _ = _LAYERS[idx]
    if idx == 5:
        gl = jnp.repeat(g, 2)
        bl = jnp.repeat(b, 2)
        L = 2 * Cout
    else:
        gl = jnp.zeros((Wo + 2, Cout)).at[1:Wo + 1].set(g).reshape(-1)
        bl = jnp.zeros((Wo + 2, Cout)).at[1:Wo + 1].set(b).reshape(-1)
        L = (Wo + 2) * Cout
    return gl.reshape(1, L), bl.reshape(1, L)


def _perm_matrix(MT):
    pm = np.zeros((MT, MT), np.float32)
    for q in range(MT // 2):
        pm[q, 2 * q] = 1.0
        pm[MT // 2 + q, 2 * q + 1] = 1.0
    return jnp.asarray(pm, dtype=_BF)


def _conv_mid_body(p0, p1, bw, gl, bl, pm, o0, o1, *, Ho, MT, Lout):
    gv = gl[...]
    bv = bl[...]
    pmv = pm[...]
    h2 = MT // 2
    for y0 in range(0, Ho, MT):
        acc = jnp.dot(p0[0, y0:y0 + MT, :], bw[0],
                      preferred_element_type=jnp.float32)
        acc = acc + jnp.dot(p1[0, y0:y0 + MT, :], bw[1],
                            preferred_element_type=jnp.float32)
        acc = acc + jnp.dot(p0[0, y0 + 1:y0 + MT + 1, :], bw[2],
                            preferred_element_type=jnp.float32)
        yb = jnp.maximum(gv * acc / _BN_DIV + bv, 0.0).astype(_BF)
        perm = jnp.dot(pmv, yb, preferred_element_type=jnp.float32).astype(_BF)
        o1[0, y0 // 2:y0 // 2 + h2, :] = perm[0:h2, :]
        o0[0, y0 // 2 + 1:y0 // 2 + 1 + h2, :] = perm[h2:MT, :]
    zrow = jnp.zeros((1, Lout), _BF)
    o0[0, 0:1, :] = zrow
    o1[0, Ho // 2:Ho // 2 + 1, :] = zrow


def _conv_last_body(p0, p1, bw, gl, bl, out):
    MT = 16
    acc = jnp.dot(p0[0, 0:MT, :], bw[0], preferred_element_type=jnp.float32)
    acc = acc + jnp.dot(p1[0, 0:MT, :], bw[1],
                        preferred_element_type=jnp.float32)
    acc = acc + jnp.dot(p0[0, 1:MT + 1, :], bw[2],
                        preferred_element_type=jnp.float32)
    out[0] = jnp.maximum(gl[...] * acc / _BN_DIV + bl[...], 0.0).astype(_BF)


def _conv_layer(p0, p1, w, g, b, idx):
    Cin, Cout, H, W, Ho, Wo, MT = _LAYERS[idx]
    B = p0.shape[0]
    U = H // 2 + 1
    Lin = (W + 2) * Cin
    bw = _banded_weights(w, idx)
    gl, bl = _bn_lanes(g, b, idx)
    pspec = pl.BlockSpec((1, U, Lin), lambda i: (i, 0, 0))
    bwspec = pl.BlockSpec(bw.shape, lambda i: (0, 0, 0))
    bnspec = pl.BlockSpec(gl.shape, lambda i: (0, 0))
    if idx == 5:
        out = pl.pallas_call(
            _conv_last_body,
            grid=(B,),
            in_specs=[pspec, pspec, bwspec, bnspec, bnspec],
            out_specs=pl.BlockSpec((1, Ho, 2 * Cout), lambda i: (i, 0, 0)),
            out_shape=jax.ShapeDtypeStruct((B, Ho, 2 * Cout), _BF),
        )(p0, p1, bw, gl, bl)
        return out, None
    Lout = (Wo + 2) * Cout
    Uo = Ho // 2 + 1
    pm = _perm_matrix(MT)
    body = functools.partial(_conv_mid_body, Ho=Ho, MT=MT, Lout=Lout)
    o0, o1 = pl.pallas_call(
        body,
        grid=(B,),
        in_specs=[pspec, pspec, bwspec, bnspec, bnspec,
                  pl.BlockSpec((MT, MT), lambda i: (0, 0))],
        out_specs=(pl.BlockSpec((1, Uo, Lout), lambda i: (i, 0, 0)),) * 2,
        out_shape=(jax.ShapeDtypeStruct((B, Uo, Lout), _BF),) * 2,
    )(p0, p1, bw, gl, bl, pm)
    return o0, o1


def _gru_vq_body(hs, wih, whh, bih, bhh, cb1, cb2, cb3,
                 zq1o, zq2o, zq3o, zsumo, i1o, losso):
    T, B, D = 16, 32, 256
    hs_v = hs[...]
    gi_all = lax.dot_general(hs_v, wih[...], (((1,), (1,)), ((), ())),
                             preferred_element_type=jnp.float32) + bih[...]
    h = jnp.zeros((B, D), jnp.float32)
    for t in range(T):
        gi = gi_all[t * B:(t + 1) * B, :]
        gh = lax.dot_general(h.astype(_BF), whh[...], (((1,), (1,)), ((), ())),
                             preferred_element_type=jnp.float32) + bhh[...]
        r = jax.nn.sigmoid(gi[:, 0:D] + gh[:, 0:D])
        z = jax.nn.sigmoid(gi[:, D:2 * D] + gh[:, D:2 * D])
        n = jnp.tanh(gi[:, 2 * D:3 * D] + r * gh[:, 2 * D:3 * D])
        h = (1.0 - z) * n + z * h

    res = h
    loss = jnp.zeros((), jnp.float32)
    outs = []
    K = 1024
    ones = jnp.ones((1, D), jnp.float32)
    ii = lax.broadcasted_iota(jnp.int32, (B, K), 1)
    for k, cb in enumerate((cb1, cb2, cb3)):
        emb = cb[...]
        embb = emb.astype(_BF)
        e2 = lax.dot_general(ones, emb * emb, (((1,), (1,)), ((), ())),
                             precision=_HIGH,
                             preferred_element_type=jnp.float32)
        z2 = jnp.sum(res * res, axis=1, keepdims=True)
        s = lax.dot_general(res.astype(_BF), embb, (((1,), (1,)), ((), ())),
                            preferred_element_type=jnp.float32)
        d = z2 + e2 - 2.0 * s
        m = jnp.min(d, axis=1, keepdims=True)
        cand = jnp.where(d == m, ii, K)
        idxc = jnp.min(cand, axis=1, keepdims=True)
        if k == 0:
            i1o[...] = idxc
        enc = (ii == idxc).astype(_BF)
        zq = lax.dot_general(enc, embb, (((1,), (0,)), ((), ())),
                             preferred_element_type=jnp.float32)
        diff = zq - res
        loss = loss + 1.25 * jnp.mean(diff * diff)
        zq = res + (zq - res)
        outs.append(zq)
        res = res - zq
    zq1o[...] = outs[0]
    zq2o[...] = outs[1]
    zq3o[...] = outs[2]
    zsumo[...] = outs[0] + outs[1] + outs[2]
    losso[...] = jnp.zeros((1, 1), jnp.float32) + loss


def kernel(speech, conv_w0, conv_w1, conv_w2, conv_w3, conv_w4, conv_w5,
           bn_g0, bn_g1, bn_g2, bn_g3, bn_g4, bn_g5,
           bn_b0, bn_b1, bn_b2, bn_b3, bn_b4, bn_b5,
           w_ih, w_hh, b_ih, b_hh, cb1, cb2, cb3):
    B = speech.shape[0]
    xp = jnp.pad(speech, ((0, 0), (1, 1), (1, 1))).astype(_BF)
    p0 = xp[:, 0::2, :]
    p1 = xp[:, 1::2, :]
    conv_ws = [conv_w0, conv_w1, conv_w2, conv_w3, conv_w4, conv_w5]
    bn_gs = [bn_g0, bn_g1, bn_g2, bn_g3, bn_g4, bn_g5]
    bn_bs = [bn_b0, bn_b1, bn_b2, bn_b3, bn_b4, bn_b5]
    for i in range(6):
        p0, p1 = _conv_layer(p0, p1, conv_ws[i], bn_gs[i], bn_bs[i], i)
    hs = p0.transpose(1, 0, 2).reshape(16 * B, 256)

    zq1, zq2, zq3, zsum, i1, lossm = pl.pallas_call(
        _gru_vq_body,
        out_shape=(
            jax.ShapeDtypeStruct((32, 256), jnp.float32),
            jax.ShapeDtypeStruct((32, 256), jnp.float32),
            jax.ShapeDtypeStruct((32, 256), jnp.float32),
            jax.ShapeDtypeStruct((32, 256), jnp.float32),
            jax.ShapeDtypeStruct((32, 1), jnp.int32),
            jax.ShapeDtypeStruct((1, 1), jnp.float32),
        ),
    )(hs, w_ih.astype(_BF), w_hh.astype(_BF),
      b_ih.reshape(1, 768), b_hh.reshape(1, 768), cb1, cb2, cb3)

    z_q_out = jnp.concatenate([zq1, zq2, zq3], axis=1)
    vq_loss = lossm[0, 0]
    codebooks = (zq1, zq2, zq3, zsum)
    return z_q_out, vq_loss, i1, codebooks


# speedup vs baseline: 9.6855x; 1.2604x over previous
import functools

import numpy as np
import jax
import jax.numpy as jnp
from jax import lax
from jax.experimental import pallas as pl

_BN_DIV = np.float32(np.sqrt(np.float32(1.0 + 1e-5)))
_HIGH = lax.Precision.HIGHEST
_BF = jnp.bfloat16

_LAYERS = [
    (1, 32, 1024, 80, 512, 40, 256),
    (32, 32, 512, 40, 256, 20, 256),
    (32, 64, 256, 20, 128, 10, 128),
    (64, 64, 128, 10, 64, 5, 64),
    (64, 128, 64, 5, 32, 3, 32),
    (128, 128, 32, 3, 16, 2, 16),
]


def _banded_weights(w, idx):
    Cin, Cout, H, W, Ho, Wo, _ = _LAYERS[idx]
    last = idx == 5
    Wi, Wout = W + 2, (2 if last else Wo + 2)
    wt = w.transpose(2, 3, 1, 0)
    sel = np.zeros((3, Wi, Wout), np.float32)
    for kw in range(3):
        for wo in range(Wo):
            sel[kw, 2 * wo + kw, wo if last else wo + 1] = 1.0
    sel = jnp.asarray(sel)
    bws = []
    for kh in range(3):
        bw4 = jnp.einsum('kio,kcd->icod', sel, wt[kh])
        if last:
            bw4 = bw4.transpose(0, 1, 3, 2)
        bws.append(bw4.reshape(Wi * Cin, Wout * Cout))
    return jnp.stack(bws, axis=0).astype(_BF)


def _bn_lanes(g, b, idx):
    Cin, Cout, H, W, Ho, Wo, _ = _LAYERS[idx]
    if idx == 5:
        gl = jnp.repeat(g, 2)
        bl = jnp.repeat(b, 2)
        L = 2 * Cout
    else:
        gl = jnp.zeros((Wo + 2, Cout)).at[1:Wo + 1].set(g).reshape(-1)
        bl = jnp.zeros((Wo + 2, Cout)).at[1:Wo + 1].set(b).reshape(-1)
        L = (Wo + 2) * Cout
    return gl.reshape(1, L), bl.reshape(1, L)


def _perm_matrix(MT):
    pm = np.zeros((MT, MT), np.float32)
    for q in range(MT // 2):
        pm[q, 2 * q] = 1.0
        pm[MT // 2 + q, 2 * q + 1] = 1.0
    return jnp.asarray(pm, dtype=_BF)


def _conv_mid_body(p0, p1, bw, gl, bl, pm, o0, o1, *, Ho, MT, Lout):
    gv = gl[...]
    bv = bl[...]
    pmv = pm[...]
    h2 = MT // 2
    for y0 in range(0, Ho, MT):
        acc = jnp.dot(p0[0, y0:y0 + MT, :], bw[0],
                      preferred_element_type=jnp.float32)
        acc = acc + jnp.dot(p1[0, y0:y0 + MT, :], bw[1],
                            preferred_element_type=jnp.float32)
        acc = acc + jnp.dot(p0[0, y0 + 1:y0 + MT + 1, :], bw[2],
                            preferred_element_type=jnp.float32)
        yb = jnp.maximum(gv * acc / _BN_DIV + bv, 0.0).astype(_BF)
        perm = jnp.dot(pmv, yb, preferred_element_type=jnp.float32).astype(_BF)
        o1[0, y0 // 2:y0 // 2 + h2, :] = perm[0:h2, :]
        o0[0, y0 // 2 + 1:y0 // 2 + 1 + h2, :] = perm[h2:MT, :]
    zrow = jnp.zeros((1, Lout), _BF)
    o0[0, 0:1, :] = zrow
    o1[0, Ho // 2:Ho // 2 + 1, :] = zrow


def _conv_u(p0v, p1v, bw, glv, blv, Ho):
    M = Ho * 32
    a = jnp.dot(p0v[0:M], bw[0], preferred_element_type=jnp.float32)
    a = a + jnp.dot(p1v[0:M], bw[1], preferred_element_type=jnp.float32)
    a = a + jnp.dot(p0v[32:M + 32], bw[2], preferred_element_type=jnp.float32)
    return jnp.maximum(glv * a / _BN_DIV + blv, 0.0).astype(_BF)


def _usplit(x, Ho, L):
    v = x.reshape(Ho // 2, 2, 32, L)
    ev = v[:, 0].reshape(Ho // 2 * 32, L)
    od = v[:, 1].reshape(Ho // 2 * 32, L)
    p0 = jnp.concatenate([jnp.zeros((32, L), _BF), od], axis=0)
    return p0, ev


def _conv_layer(p0, p1, w, g, b, idx):
    Cin, Cout, H, W, Ho, Wo, MT = _LAYERS[idx]
    B = p0.shape[0]
    U = H // 2 + 1
    Lin = (W + 2) * Cin
    bw = _banded_weights(w, idx)
    gl, bl = _bn_lanes(g, b, idx)
    pspec = pl.BlockSpec((1, U, Lin), lambda i: (i, 0, 0))
    bwspec = pl.BlockSpec(bw.shape, lambda i: (0, 0, 0))
    bnspec = pl.BlockSpec(gl.shape, lambda i: (0, 0))
    Lout = (Wo + 2) * Cout
    Uo = Ho // 2 + 1
    pm = _perm_matrix(MT)
    body = functools.partial(_conv_mid_body, Ho=Ho, MT=MT, Lout=Lout)
    o0, o1 = pl.pallas_call(
        body,
        grid=(B,),
        in_specs=[pspec, pspec, bwspec, bnspec, bnspec,
                  pl.BlockSpec((MT, MT), lambda i: (0, 0))],
        out_specs=(pl.BlockSpec((1, Uo, Lout), lambda i: (i, 0, 0)),) * 2,
        out_shape=(jax.ShapeDtypeStruct((B, Uo, Lout), _BF),) * 2,
    )(p0, p1, bw, gl, bl, pm)
    return o0, o1


def _tail_body(p0, p1, bw3, gl3, bl3, bw4, gl4, bl4, bw5, gl5, bl5,
               wih, whh, bih, bhh, cb1, cb2, cb3,
               zq1o, zq2o, zq3o, zsumo, i1o, losso):
    x3 = _conv_u(p0[...], p1[...], bw3, gl3[...], bl3[...], 64)
    p0_4, p1_4 = _usplit(x3, 64, 448)
    x4 = _conv_u(p0_4, p1_4, bw4, gl4[...], bl4[...], 32)
    p0_5, p1_5 = _usplit(x4, 32, 640)
    hs_v = _conv_u(p0_5, p1_5, bw5, gl5[...], bl5[...], 16)

    T, B, D = 16, 32, 256
    gi_all = lax.dot_general(hs_v, wih[...], (((1,), (1,)), ((), ())),
                             preferred_element_type=jnp.float32) + bih[...]
    h = jnp.zeros((B, D), jnp.float32)
    for t in range(T):
        gi = gi_all[t * B:(t + 1) * B, :]
        gh = lax.dot_general(h.astype(_BF), whh[...], (((1,), (1,)), ((), ())),
                             preferred_element_type=jnp.float32) + bhh[...]
        r = jax.nn.sigmoid(gi[:, 0:D] + gh[:, 0:D])
        z = jax.nn.sigmoid(gi[:, D:2 * D] + gh[:, D:2 * D])
        n = jnp.tanh(gi[:, 2 * D:3 * D] + r * gh[:, 2 * D:3 * D])
        h = (1.0 - z) * n + z * h

    res = h
    loss = jnp.zeros((), jnp.float32)
    outs = []
    K = 1024
    ones = jnp.ones((1, D), jnp.float32)
    ii = lax.broadcasted_iota(jnp.int32, (B, K), 1)
    for k, cb in enumerate((cb1, cb2, cb3)):
        emb = cb[...]
        embb = emb.astype(_BF)
        e2 = lax.dot_general(ones, emb * emb, (((1,), (1,)), ((), ())),
                             precision=_HIGH,
                             preferred_element_type=jnp.float32)
        z2 = jnp.sum(res * res, axis=1, keepdims=True)
        s = lax.dot_general(res.astype(_BF), embb, (((1,), (1,)), ((), ())),
                            preferred_element_type=jnp.float32)
        d = z2 + e2 - 2.0 * s
        m = jnp.min(d, axis=1, keepdims=True)
        cand = jnp.where(d == m, ii, K)
        idxc = jnp.min(cand, axis=1, keepdims=True)
        if k == 0:
            i1o[...] = idxc
        enc = (ii == idxc).astype(_BF)
        zq = lax.dot_general(enc, embb, (((1,), (0,)), ((), ())),
                             preferred_element_type=jnp.float32)
        diff = zq - res
        loss = loss + 1.25 * jnp.mean(diff * diff)
        zq = res + (zq - res)
        outs.append(zq)
        res = res - zq
    zq1o[...] = outs[0]
    zq2o[...] = outs[1]
    zq3o[...] = outs[2]
    zsumo[...] = outs[0] + outs[1] + outs[2]
    losso[...] = jnp.zeros((1, 1), jnp.float32) + loss


def kernel(speech, conv_w0, conv_w1, conv_w2, conv_w3, conv_w4, conv_w5,
           bn_g0, bn_g1, bn_g2, bn_g3, bn_g4, bn_g5,
           bn_b0, bn_b1, bn_b2, bn_b3, bn_b4, bn_b5,
           w_ih, w_hh, b_ih, b_hh, cb1, cb2, cb3):
    B = speech.shape[0]
    xp = jnp.pad(speech, ((0, 0), (1, 1), (1, 1))).astype(_BF)
    p0 = xp[:, 0::2, :]
    p1 = xp[:, 1::2, :]
    conv_ws = [conv_w0, conv_w1, conv_w2, conv_w3, conv_w4, conv_w5]
    bn_gs = [bn_g0, bn_g1, bn_g2, bn_g3, bn_g4, bn_g5]
    bn_bs = [bn_b0, bn_b1, bn_b2, bn_b3, bn_b4, bn_b5]
    for i in range(3):
        p0, p1 = _conv_layer(p0, p1, conv_ws[i], bn_gs[i], bn_bs[i], i)
    p0u = p0.transpose(1, 0, 2).reshape(65 * B, 768)
    p1u = p1.transpose(1, 0, 2).reshape(65 * B, 768)
    tail_w = []
    for i in (3, 4, 5):
        tail_w.append(_banded_weights(conv_ws[i], i))
        tail_w.extend(_bn_lanes(bn_gs[i], bn_bs[i], i))

    zq1, zq2, zq3, zsum, i1, lossm = pl.pallas_call(
        _tail_body,
        out_shape=(
            jax.ShapeDtypeStruct((32, 256), jnp.float32),
            jax.ShapeDtypeStruct((32, 256), jnp.float32),
            jax.ShapeDtypeStruct((32, 256), jnp.float32),
            jax.ShapeDtypeStruct((32, 256), jnp.float32),
            jax.ShapeDtypeStruct((32, 1), jnp.int32),
            jax.ShapeDtypeStruct((1, 1), jnp.float32),
        ),
    )(p0u, p1u, *tail_w, w_ih.astype(_BF), w_hh.astype(_BF),
      b_ih.reshape(1, 768), b_hh.reshape(1, 768), cb1, cb2, cb3)

    z_q_out = jnp.concatenate([zq1, zq2, zq3], axis=1)
    vq_loss = lossm[0, 0]
    codebooks = (zq1, zq2, zq3, zsum)
    return z_q_out, vq_loss, i1, codebooks


# speedup vs baseline: 10.2443x; 1.0577x over previous
import functools

import numpy as np
import jax
import jax.numpy as jnp
from jax import lax
from jax.experimental import pallas as pl

_BN_DIV = np.float32(np.sqrt(np.float32(1.0 + 1e-5)))
_HIGH = lax.Precision.HIGHEST
_BF = jnp.bfloat16

_LAYERS = [
    (1, 32, 1024, 80, 512, 40, 256),
    (32, 32, 512, 40, 256, 20, 256),
    (32, 64, 256, 20, 128, 10, 128),
    (64, 64, 128, 10, 64, 5, 64),
    (64, 128, 64, 5, 32, 3, 32),
    (128, 128, 32, 3, 16, 2, 16),
]


def _banded_weights(w, idx):
    Cin, Cout, H, W, Ho, Wo, _ = _LAYERS[idx]
    last = idx == 5
    Wi, Wout = W + 2, (2 if last else Wo + 2)
    wt = w.transpose(2, 3, 1, 0)
    sel = np.zeros((3, Wi, Wout), np.float32)
    for kw in range(3):
        for wo in range(Wo):
            sel[kw, 2 * wo + kw, wo if last else wo + 1] = 1.0
    sel = jnp.asarray(sel)
    bws = []
    for kh in range(3):
        bw4 = jnp.einsum('kio,kcd->icod', sel, wt[kh])
        if last:
            bw4 = bw4.transpose(0, 1, 3, 2)
        bws.append(bw4.reshape(Wi * Cin, Wout * Cout))
    return jnp.stack(bws, axis=0).astype(_BF)


def _bn_lanes(g, b, idx):
    Cin, Cout, H, W, Ho, Wo, _ = _LAYERS[idx]
    if idx == 5:
        gl = jnp.repeat(g, 2)
        bl = jnp.repeat(b, 2)
        L = 2 * Cout
    else:
        gl = jnp.zeros((Wo + 2, Cout)).at[1:Wo + 1].set(g).reshape(-1)
        bl = jnp.zeros((Wo + 2, Cout)).at[1:Wo + 1].set(b).reshape(-1)
        L = (Wo + 2) * Cout
    return gl.reshape(1, L), bl.reshape(1, L)


def _perm_matrix(MT):
    pm = np.zeros((MT, MT), np.float32)
    for q in range(MT // 2):
        pm[q, 2 * q] = 1.0
        pm[MT // 2 + q, 2 * q + 1] = 1.0
    return jnp.asarray(pm, dtype=_BF)


_GROUPS = {
    1: ((0, 512, 0, 256), (384, 640, 256, 256), (896, 448, 512, 192)),
    2: ((0, 256, 0, 256), (128, 384, 256, 256), (384, 320, 512, 256)),
}


def _make_mid_body(Ho, MT, Lout, groups):
    ngrp = len(groups)

    def body(p0, p1, *rest):
        bws = rest[0:ngrp]
        gl, bl, pm = rest[ngrp:ngrp + 3]
        o0, o1 = rest[ngrp + 3:ngrp + 5]
        pmv = pm[...]
        h2 = MT // 2
        for y0 in range(0, Ho, MT):
            parts = []
            for bwg, (r0, K, c0, N) in zip(bws, groups):
                acc = jnp.dot(p0[0, y0:y0 + MT, r0:r0 + K], bwg[0],
                              preferred_element_type=jnp.float32)
                acc = acc + jnp.dot(p1[0, y0:y0 + MT, r0:r0 + K], bwg[1],
                                    preferred_element_type=jnp.float32)
                acc = acc + jnp.dot(p0[0, y0 + 1:y0 + MT + 1, r0:r0 + K],
                                    bwg[2], preferred_element_type=jnp.float32)
                y = gl[0:1, c0:c0 + N] * acc / _BN_DIV + bl[0:1, c0:c0 + N]
                parts.append(jnp.maximum(y, 0.0).astype(_BF))
            yb = parts[0] if len(parts) == 1 else jnp.concatenate(parts, axis=1)
            perm = jnp.dot(pmv, yb,
                           preferred_element_type=jnp.float32).astype(_BF)
            o1[0, y0 // 2:y0 // 2 + h2, :] = perm[0:h2, :]
            o0[0, y0 // 2 + 1:y0 // 2 + 1 + h2, :] = perm[h2:MT, :]
        zrow = jnp.zeros((1, Lout), _BF)
        o0[0, 0:1, :] = zrow
        o1[0, Ho // 2:Ho // 2 + 1, :] = zrow

    return body


def _conv_u(p0v, p1v, bw, glv, blv, Ho):
    M = Ho * 32
    a = jnp.dot(p0v[0:M], bw[0], preferred_element_type=jnp.float32)
    a = a + jnp.dot(p1v[0:M], bw[1], preferred_element_type=jnp.float32)
    a = a + jnp.dot(p0v[32:M + 32], bw[2], preferred_element_type=jnp.float32)
    return jnp.maximum(glv * a / _BN_DIV + blv, 0.0).astype(_BF)


def _usplit(x, Ho, L):
    v = x.reshape(Ho // 2, 2, 32, L)
    ev = v[:, 0].reshape(Ho // 2 * 32, L)
    od = v[:, 1].reshape(Ho // 2 * 32, L)
    p0 = jnp.concatenate([jnp.zeros((32, L), _BF), od], axis=0)
    return p0, ev


def _conv_layer(p0, p1, w, g, b, idx):
    Cin, Cout, H, W, Ho, Wo, MT = _LAYERS[idx]
    B = p0.shape[0]
    U = H // 2 + 1
    Lin = (W + 2) * Cin
    bw = _banded_weights(w, idx)
    gl, bl = _bn_lanes(g, b, idx)
    pspec = pl.BlockSpec((1, U, Lin), lambda i: (i, 0, 0))
    bnspec = pl.BlockSpec(gl.shape, lambda i: (0, 0))
    Lout = (Wo + 2) * Cout
    Uo = Ho // 2 + 1
    pm = _perm_matrix(MT)
    groups = _GROUPS.get(idx, ((0, Lin, 0, Lout),))
    bws = [bw[:, r0:r0 + K, c0:c0 + N] for (r0, K, c0, N) in groups]
    body = _make_mid_body(Ho, MT, Lout, groups)
    o0, o1 = pl.pallas_call(
        body,
        grid=(B,),
        in_specs=[pspec, pspec]
        + [pl.BlockSpec(bg.shape, lambda i: (0, 0, 0)) for bg in bws]
        + [bnspec, bnspec, pl.BlockSpec((MT, MT), lambda i: (0, 0))],
        out_specs=(pl.BlockSpec((1, Uo, Lout), lambda i: (i, 0, 0)),) * 2,
        out_shape=(jax.ShapeDtypeStruct((B, Uo, Lout), _BF),) * 2,
    )(p0, p1, *bws, gl, bl, pm)
    return o0, o1


def _tail_body(p0, p1, bw3, gl3, bl3, bw4, gl4, bl4, bw5, gl5, bl5,
               wih, whh, bih, bhh, cb1, cb2, cb3,
               zq1o, zq2o, zq3o, zsumo, i1o, losso):
    x3 = _conv_u(p0[...], p1[...], bw3, gl3[...], bl3[...], 64)
    p0_4, p1_4 = _usplit(x3, 64, 448)
    x4 = _conv_u(p0_4, p1_4, bw4, gl4[...], bl4[...], 32)
    p0_5, p1_5 = _usplit(x4, 32, 640)
    hs_v = _conv_u(p0_5, p1_5, bw5, gl5[...], bl5[...], 16)

    T, B, D = 16, 32, 256
    gi_all = lax.dot_general(hs_v, wih[...], (((1,), (1,)), ((), ())),
                             preferred_element_type=jnp.float32) + bih[...]
    h = jnp.zeros((B, D), jnp.float32)
    for t in range(T):
        gi = gi_all[t * B:(t + 1) * B, :]
        gh = lax.dot_general(h.astype(_BF), whh[...], (((1,), (1,)), ((), ())),
                             preferred_element_type=jnp.float32) + bhh[...]
        r = jax.nn.sigmoid(gi[:, 0:D] + gh[:, 0:D])
        z = jax.nn.sigmoid(gi[:, D:2 * D] + gh[:, D:2 * D])
        n = jnp.tanh(gi[:, 2 * D:3 * D] + r * gh[:, 2 * D:3 * D])
        h = (1.0 - z) * n + z * h

    res = h
    loss = jnp.zeros((), jnp.float32)
    outs = []
    K = 1024
    ones = jnp.ones((1, D), jnp.float32)
    ii = lax.broadcasted_iota(jnp.int32, (B, K), 1)
    for k, cb in enumerate((cb1, cb2, cb3)):
        emb = cb[...]
        embb = emb.astype(_BF)
        e2 = lax.dot_general(ones, emb * emb, (((1,), (1,)), ((), ())),
                             precision=_HIGH,
                             preferred_element_type=jnp.float32)
        z2 = jnp.sum(res * res, axis=1, keepdims=True)
        s = lax.dot_general(res.astype(_BF), embb, (((1,), (1,)), ((), ())),
                            preferred_element_type=jnp.float32)
        d = z2 + e2 - 2.0 * s
        m = jnp.min(d, axis=1, keepdims=True)
        cand = jnp.where(d == m, ii, K)
        idxc = jnp.min(cand, axis=1, keepdims=True)
        if k == 0:
            i1o[...] = idxc
        enc = (ii == idxc).astype(_BF)
        zq = lax.dot_general(enc, embb, (((1,), (0,)), ((), ())),
                             preferred_element_type=jnp.float32)
        diff = zq - res
        loss = loss + 1.25 * jnp.mean(diff * diff)
        zq = res + (zq - res)
        outs.append(zq)
        res = res - zq
    zq1o[...] = outs[0]
    zq2o[...] = outs[1]
    zq3o[...] = outs[2]
    zsumo[...] = outs[0] + outs[1] + outs[2]
    losso[...] = jnp.zeros((1, 1), jnp.float32) + loss


def kernel(speech, conv_w0, conv_w1, conv_w2, conv_w3, conv_w4, conv_w5,
           bn_g0, bn_g1, bn_g2, bn_g3, bn_g4, bn_g5,
           bn_b0, bn_b1, bn_b2, bn_b3, bn_b4, bn_b5,
           w_ih, w_hh, b_ih, b_hh, cb1, cb2, cb3):
    B = speech.shape[0]
    xp = jnp.pad(speech, ((0, 0), (1, 1), (1, 1))).astype(_BF)
    p0 = xp[:, 0::2, :]
    p1 = xp[:, 1::2, :]
    conv_ws = [conv_w0, conv_w1, conv_w2, conv_w3, conv_w4, conv_w5]
    bn_gs = [bn_g0, bn_g1, bn_g2, bn_g3, bn_g4, bn_g5]
    bn_bs = [bn_b0, bn_b1, bn_b2, bn_b3, bn_b4, bn_b5]
    for i in range(3):
        p0, p1 = _conv_layer(p0, p1, conv_ws[i], bn_gs[i], bn_bs[i], i)
    p0u = p0.transpose(1, 0, 2).reshape(65 * B, 768)
    p1u = p1.transpose(1, 0, 2).reshape(65 * B, 768)
    tail_w = []
    for i in (3, 4, 5):
        tail_w.append(_banded_weights(conv_ws[i], i))
        tail_w.extend(_bn_lanes(bn_gs[i], bn_bs[i], i))

    zq1, zq2, zq3, zsum, i1, lossm = pl.pallas_call(
        _tail_body,
        out_shape=(
            jax.ShapeDtypeStruct((32, 256), jnp.float32),
            jax.ShapeDtypeStruct((32, 256), jnp.float32),
            jax.ShapeDtypeStruct((32, 256), jnp.float32),
            jax.ShapeDtypeStruct((32, 256), jnp.float32),
            jax.ShapeDtypeStruct((32, 1), jnp.int32),
            jax.ShapeDtypeStruct((1, 1), jnp.float32),
        ),
    )(p0u, p1u, *tail_w, w_ih.astype(_BF), w_hh.astype(_BF),
      b_ih.reshape(1, 768), b_hh.reshape(1, 768), cb1, cb2, cb3)

    z_q_out = jnp.concatenate([zq1, zq2, zq3], axis=1)
    vq_loss = lossm[0, 0]
    codebooks = (zq1, zq2, zq3, zsum)
    return z_q_out, vq_loss, i1, codebooks
